# Initial kernel scaffold; baseline (speedup 1.0000x reference)
import jax
import jax.numpy as jnp
from jax.experimental import pallas as pl


def kernel(x, br_gn_g, br_gn_b, br_wih, br_bih, br_whh, br_wp_f, br_wp_b, br_bp, bc_gn_g, bc_gn_b, bc_wih, bc_bih, bc_whh, bc_wp_f, bc_wp_b, bc_bp):
    raise NotImplementedError("write your pallas kernel here")


# speedup vs baseline: 4.4146x; 4.4146x over previous
import functools

import jax
import jax.numpy as jnp
from jax import lax
from jax.experimental import pallas as pl
from jax.experimental.pallas import tpu as pltpu

EPS = float(jnp.finfo(jnp.float32).eps)

MAX_LTB = 16384


def _row_block(bp, l):
    cap = max(8, MAX_LTB // max(l, 1))
    tb = min(bp // 2, cap)
    while tb > 8:
        if bp % tb == 0 and tb % 8 == 0:
            return tb
        tb -= 8
    return min(bp, 8)


def _resrnn_kernel(x_ref, g_ref, b_ref, wihf_ref, wihb_ref, bihf_ref, bihb_ref,
                   whhf_ref, whhb_ref, wpf_ref, wpb_ref, bp_ref, o_ref,
                   xn_sc):
    L, TB, C = x_ref.shape
    H = wpf_ref.shape[0]

    x = x_ref[...]
    inv_n = 1.0 / (L * C)
    mean = jnp.sum(jnp.sum(x, axis=2, keepdims=True), axis=0, keepdims=True) * inv_n
    d = x - mean
    var = jnp.sum(jnp.sum(d * d, axis=2, keepdims=True), axis=0, keepdims=True) * inv_n
    xn_sc[...] = (d * lax.rsqrt(var + EPS) * g_ref[...] + b_ref[...]).astype(jnp.bfloat16)

    o_ref[...] = x + bp_ref[...]

    wih_f = wihf_ref[...]
    wih_b = wihb_ref[...]
    bih_f = bihf_ref[...]
    bih_b = bihb_ref[...]
    whh_f = whhf_ref[...]
    whh_b = whhb_ref[...]
    wp_f = wpf_ref[...]
    wp_b = wpb_ref[...]

    def gate(g, c_prev):
        i = jax.nn.sigmoid(g[:, 0:H])
        f = jax.nn.sigmoid(g[:, H:2 * H])
        z = jnp.tanh(g[:, 2 * H:3 * H])
        o = jax.nn.sigmoid(g[:, 3 * H:4 * H])
        c = f * c_prev + i * z
        return o * jnp.tanh(c), c

    def step(t, carry):
        h_f, c_f, h_b, c_b = carry
        pre_f = (jnp.dot(xn_sc[t], wih_f, preferred_element_type=jnp.float32)
                 + bih_f
                 + jnp.dot(h_f.astype(jnp.bfloat16), whh_f,
                           preferred_element_type=jnp.float32))
        pre_b = (jnp.dot(xn_sc[L - 1 - t], wih_b, preferred_element_type=jnp.float32)
                 + bih_b
                 + jnp.dot(h_b.astype(jnp.bfloat16), whh_b,
                           preferred_element_type=jnp.float32))
        h_f, c_f = gate(pre_f, c_f)
        h_b, c_b = gate(pre_b, c_b)
        o_ref[t] += jnp.dot(h_f.astype(jnp.bfloat16), wp_f,
                            preferred_element_type=jnp.float32)
        o_ref[L - 1 - t] += jnp.dot(h_b.astype(jnp.bfloat16), wp_b,
                                    preferred_element_type=jnp.float32)
        return h_f, c_f, h_b, c_b

    zeros = jnp.zeros((TB, H), jnp.float32)
    lax.fori_loop(0, L, step, (zeros, zeros, zeros, zeros), unroll=2)


def _res_rnn(xt, gn_g, gn_b, wih, bih, whh, wp_f, wp_b, bp):
    L, Bp, C = xt.shape
    H = wp_f.shape[0]
    G4 = 4 * H
    TB = _row_block(Bp, L)
    grid = (Bp // TB,)

    wih_f, wih_b = wih[:, :G4], wih[:, G4:]
    bih_f, bih_b = bih[:, :G4], bih[:, G4:]
    whh_f, whh_b = whh[:H, :G4], whh[H:, G4:]

    call = pl.pallas_call(
        _resrnn_kernel,
        grid=grid,
        in_specs=[
            pl.BlockSpec((L, TB, C), lambda i: (0, i, 0)),
            pl.BlockSpec((1, 1, C), lambda i: (0, 0, 0)),
            pl.BlockSpec((1, 1, C), lambda i: (0, 0, 0)),
            pl.BlockSpec((C, G4), lambda i: (0, 0)),
            pl.BlockSpec((C, G4), lambda i: (0, 0)),
            pl.BlockSpec((1, G4), lambda i: (0, 0)),
            pl.BlockSpec((1, G4), lambda i: (0, 0)),
            pl.BlockSpec((H, G4), lambda i: (0, 0)),
            pl.BlockSpec((H, G4), lambda i: (0, 0)),
            pl.BlockSpec((H, C), lambda i: (0, 0)),
            pl.BlockSpec((H, C), lambda i: (0, 0)),
            pl.BlockSpec((1, C), lambda i: (0, 0)),
        ],
        out_specs=pl.BlockSpec((L, TB, C), lambda i: (0, i, 0)),
        out_shape=jax.ShapeDtypeStruct((L, Bp, C), jnp.float32),
        input_output_aliases={0: 0},

        scratch_shapes=[
            pltpu.VMEM((L, TB, C), jnp.bfloat16),
        ],
        compiler_params=pltpu.CompilerParams(
            dimension_semantics=("parallel",)),
    )
    return call(xt, gn_g, gn_b, wih_f, wih_b, bih_f, bih_b,
                whh_f, whh_b, wp_f, wp_b, bp)


@functools.partial(jax.jit, static_argnames=())
def kernel(x, br_gn_g, br_gn_b, br_wih, br_bih, br_whh, br_wp_f, br_wp_b, br_bp,
           bc_gn_g, bc_gn_b, bc_wih, bc_bih, bc_whh, bc_wp_f, bc_wp_b, bc_bp):
    B, N, T = x.shape
    C = br_wp_f.shape[1]
    nband = N // C
    xt = jnp.transpose(x.reshape(B, nband, C, T), (3, 0, 1, 2)).reshape(T, B * nband, C)
    y1 = _res_rnn(xt, br_gn_g, br_gn_b, br_wih, br_bih, br_whh,
                  br_wp_f, br_wp_b, br_bp)
    xb = jnp.transpose(y1.reshape(T, B, nband, C), (2, 1, 0, 3)).reshape(nband, B * T, C)
    y2 = _res_rnn(xb, bc_gn_g, bc_gn_b, bc_wih, bc_bih, bc_whh,
                  bc_wp_f, bc_wp_b, bc_bp)
    return jnp.transpose(y2.reshape(nband, B, T, C), (1, 0, 3, 2)).reshape(B, N, T)


# speedup vs baseline: 4.9184x; 1.1141x over previous
import functools

import jax
import jax.numpy as jnp
from jax import lax
from jax.experimental import pallas as pl
from jax.experimental.pallas import tpu as pltpu

EPS = float(jnp.finfo(jnp.float32).eps)

MAX_LTB = 16384


def _row_block(bp, l):
    cap = max(8, MAX_LTB // max(l, 1))
    tb = min(bp // 2, cap)
    while tb > 8:
        if bp % tb == 0 and tb % 8 == 0:
            return tb
        tb -= 8
    return min(bp, 8)


def _resrnn_kernel(x_ref, g_ref, b_ref, wihf_ref, wihb_ref, bihf_ref, bihb_ref,
                   whhf_ref, whhb_ref, wpf_ref, wpb_ref, bp_ref, o_ref,
                   xn_sc):
    L, TB, C = x_ref.shape
    H = wpf_ref.shape[0]

    x = x_ref[...]
    inv_n = 1.0 / (L * C)
    s = jnp.sum(x, axis=0, keepdims=True)
    mean = jnp.sum(s, axis=2, keepdims=True) * inv_n
    d = x - mean
    v = jnp.sum(d * d, axis=0, keepdims=True)
    var = jnp.sum(v, axis=2, keepdims=True) * inv_n
    xn_sc[...] = (d * lax.rsqrt(var + EPS) * g_ref[...] + b_ref[...]).astype(jnp.bfloat16)

    o_ref[...] = x + bp_ref[...]

    wih_f = wihf_ref[...]
    wih_b = wihb_ref[...]
    bih_f = bihf_ref[...]
    bih_b = bihb_ref[...]
    whh_f = whhf_ref[...]
    whh_b = whhb_ref[...]
    wp_f = wpf_ref[...]
    wp_b = wpb_ref[...]

    def gate(g, c_prev):
        i = jax.nn.sigmoid(g[:, 0:H])
        f = jax.nn.sigmoid(g[:, H:2 * H])
        z = jnp.tanh(g[:, 2 * H:3 * H])
        o = jax.nn.sigmoid(g[:, 3 * H:4 * H])
        c = f * c_prev + i * z
        return o * jnp.tanh(c), c

    def step(t, carry):
        h_f, c_f, h_b, c_b = carry
        pre_f = (jnp.dot(xn_sc[t], wih_f, preferred_element_type=jnp.float32)
                 + bih_f
                 + jnp.dot(h_f.astype(jnp.bfloat16), whh_f,
                           preferred_element_type=jnp.float32))
        pre_b = (jnp.dot(xn_sc[L - 1 - t], wih_b, preferred_element_type=jnp.float32)
                 + bih_b
                 + jnp.dot(h_b.astype(jnp.bfloat16), whh_b,
                           preferred_element_type=jnp.float32))
        h_f, c_f = gate(pre_f, c_f)
        h_b, c_b = gate(pre_b, c_b)
        o_ref[t] += jnp.dot(h_f.astype(jnp.bfloat16), wp_f,
                            preferred_element_type=jnp.float32)
        o_ref[L - 1 - t] += jnp.dot(h_b.astype(jnp.bfloat16), wp_b,
                                    preferred_element_type=jnp.float32)
        return h_f, c_f, h_b, c_b

    zeros = jnp.zeros((TB, H), jnp.float32)
    lax.fori_loop(0, L, step, (zeros, zeros, zeros, zeros), unroll=4)


def _res_rnn(xt, gn_g, gn_b, wih, bih, whh, wp_f, wp_b, bp):
    L, Bp, C = xt.shape
    H = wp_f.shape[0]
    G4 = 4 * H
    TB = _row_block(Bp, L)
    grid = (Bp // TB,)

    wih_f, wih_b = wih[:, :G4], wih[:, G4:]
    bih_f, bih_b = bih[:, :G4], bih[:, G4:]
    whh_f, whh_b = whh[:H, :G4], whh[H:, G4:]

    call = pl.pallas_call(
        _resrnn_kernel,
        grid=grid,
        in_specs=[
            pl.BlockSpec((L, TB, C), lambda i: (0, i, 0)),
            pl.BlockSpec((1, 1, C), lambda i: (0, 0, 0)),
            pl.BlockSpec((1, 1, C), lambda i: (0, 0, 0)),
            pl.BlockSpec((C, G4), lambda i: (0, 0)),
            pl.BlockSpec((C, G4), lambda i: (0, 0)),
            pl.BlockSpec((1, G4), lambda i: (0, 0)),
            pl.BlockSpec((1, G4), lambda i: (0, 0)),
            pl.BlockSpec((H, G4), lambda i: (0, 0)),
            pl.BlockSpec((H, G4), lambda i: (0, 0)),
            pl.BlockSpec((H, C), lambda i: (0, 0)),
            pl.BlockSpec((H, C), lambda i: (0, 0)),
            pl.BlockSpec((1, C), lambda i: (0, 0)),
        ],
        out_specs=pl.BlockSpec((L, TB, C), lambda i: (0, i, 0)),
        out_shape=jax.ShapeDtypeStruct((L, Bp, C), jnp.float32),
        input_output_aliases={0: 0},

        scratch_shapes=[
            pltpu.VMEM((L, TB, C), jnp.bfloat16),
        ],
        compiler_params=pltpu.CompilerParams(
            dimension_semantics=("parallel",)),
    )
    return call(xt, gn_g, gn_b, wih_f, wih_b, bih_f, bih_b,
                whh_f, whh_b, wp_f, wp_b, bp)


@functools.partial(jax.jit, static_argnames=())
def kernel(x, br_gn_g, br_gn_b, br_wih, br_bih, br_whh, br_wp_f, br_wp_b, br_bp,
           bc_gn_g, bc_gn_b, bc_wih, bc_bih, bc_whh, bc_wp_f, bc_wp_b, bc_bp):
    B, N, T = x.shape
    C = br_wp_f.shape[1]
    nband = N // C
    xt = jnp.transpose(x.reshape(B, nband, C, T), (3, 0, 1, 2)).reshape(T, B * nband, C)
    y1 = _res_rnn(xt, br_gn_g, br_gn_b, br_wih, br_bih, br_whh,
                  br_wp_f, br_wp_b, br_bp)
    xb = jnp.transpose(y1.reshape(T, B, nband, C), (2, 1, 0, 3)).reshape(nband, B * T, C)
    y2 = _res_rnn(xb, bc_gn_g, bc_gn_b, bc_wih, bc_bih, bc_whh,
                  bc_wp_f, bc_wp_b, bc_bp)
    return jnp.transpose(y2.reshape(nband, B, T, C), (1, 0, 3, 2)).reshape(B, N, T)


# speedup vs baseline: 5.3454x; 1.0868x over previous
import functools

import jax
import jax.numpy as jnp
from jax import lax
from jax.experimental import pallas as pl
from jax.experimental.pallas import tpu as pltpu

EPS = float(jnp.finfo(jnp.float32).eps)

MAX_LTB = 16384


def _row_block(bp, l):
    cap = max(8, MAX_LTB // max(l, 1))
    tb = min(bp // 2, cap)
    while tb > 8:
        if bp % tb == 0 and tb % 8 == 0:
            return tb
        tb -= 8
    return min(bp, 8)


def _resrnn_kernel(x_ref, g_ref, b_ref, wf_ref, wb_ref, bihf_ref, bihb_ref,
                   wpf_ref, wpb_ref, bp_ref, o_ref, xn_sc):
    L, TB, C = x_ref.shape
    H = wpf_ref.shape[0]

    x = x_ref[...]
    inv_n = 1.0 / (L * C)
    s = jnp.sum(x, axis=0, keepdims=True)
    mean = jnp.sum(s, axis=2, keepdims=True) * inv_n
    d = x - mean
    v = jnp.sum(d * d, axis=0, keepdims=True)
    var = jnp.sum(v, axis=2, keepdims=True) * inv_n
    xn_sc[...] = (d * lax.rsqrt(var + EPS) * g_ref[...] + b_ref[...]).astype(jnp.bfloat16)

    o_ref[...] = x + bp_ref[...]

    w_f = wf_ref[...]
    w_b = wb_ref[...]
    bih_f = bihf_ref[...]
    bih_b = bihb_ref[...]
    wp_f = wpf_ref[...]
    wp_b = wpb_ref[...]

    def sig(v):
        return 0.5 * jnp.tanh(0.5 * v) + 0.5

    def gate(g, c_prev):
        i = sig(g[:, 0:H])
        f = sig(g[:, H:2 * H])
        z = jnp.tanh(g[:, 2 * H:3 * H])
        o = sig(g[:, 3 * H:4 * H])
        c = f * c_prev + i * z
        return o * jnp.tanh(c), c

    def step(t, carry):
        h_f, c_f, h_b, c_b = carry
        zf = jnp.concatenate([xn_sc[t], h_f.astype(jnp.bfloat16)], axis=1)
        zb = jnp.concatenate([xn_sc[L - 1 - t], h_b.astype(jnp.bfloat16)], axis=1)
        pre_f = jnp.dot(zf, w_f, preferred_element_type=jnp.float32) + bih_f
        pre_b = jnp.dot(zb, w_b, preferred_element_type=jnp.float32) + bih_b
        h_f, c_f = gate(pre_f, c_f)
        h_b, c_b = gate(pre_b, c_b)
        o_ref[t] += jnp.dot(h_f.astype(jnp.bfloat16), wp_f,
                            preferred_element_type=jnp.float32)
        o_ref[L - 1 - t] += jnp.dot(h_b.astype(jnp.bfloat16), wp_b,
                                    preferred_element_type=jnp.float32)
        return h_f, c_f, h_b, c_b

    zeros = jnp.zeros((TB, H), jnp.float32)
    lax.fori_loop(0, L, step, (zeros, zeros, zeros, zeros), unroll=4)


def _res_rnn(xt, gn_g, gn_b, wih, bih, whh, wp_f, wp_b, bp):
    L, Bp, C = xt.shape
    H = wp_f.shape[0]
    G4 = 4 * H
    TB = _row_block(Bp, L)
    grid = (Bp // TB,)

    w_f = jnp.concatenate([wih[:, :G4], whh[:H, :G4]], axis=0)
    w_b = jnp.concatenate([wih[:, G4:], whh[H:, G4:]], axis=0)
    bih_f, bih_b = bih[:, :G4], bih[:, G4:]

    call = pl.pallas_call(
        _resrnn_kernel,
        grid=grid,
        in_specs=[
            pl.BlockSpec((L, TB, C), lambda i: (0, i, 0)),
            pl.BlockSpec((1, 1, C), lambda i: (0, 0, 0)),
            pl.BlockSpec((1, 1, C), lambda i: (0, 0, 0)),
            pl.BlockSpec((C + H, G4), lambda i: (0, 0)),
            pl.BlockSpec((C + H, G4), lambda i: (0, 0)),
            pl.BlockSpec((1, G4), lambda i: (0, 0)),
            pl.BlockSpec((1, G4), lambda i: (0, 0)),
            pl.BlockSpec((H, C), lambda i: (0, 0)),
            pl.BlockSpec((H, C), lambda i: (0, 0)),
            pl.BlockSpec((1, C), lambda i: (0, 0)),
        ],
        out_specs=pl.BlockSpec((L, TB, C), lambda i: (0, i, 0)),
        out_shape=jax.ShapeDtypeStruct((L, Bp, C), jnp.float32),
        input_output_aliases={0: 0},

        scratch_shapes=[
            pltpu.VMEM((L, TB, C), jnp.bfloat16),
        ],
        compiler_params=pltpu.CompilerParams(
            dimension_semantics=("parallel",)),
    )
    return call(xt, gn_g, gn_b, w_f, w_b, bih_f, bih_b, wp_f, wp_b, bp)


@functools.partial(jax.jit, static_argnames=())
def kernel(x, br_gn_g, br_gn_b, br_wih, br_bih, br_whh, br_wp_f, br_wp_b, br_bp,
           bc_gn_g, bc_gn_b, bc_wih, bc_bih, bc_whh, bc_wp_f, bc_wp_b, bc_bp):
    B, N, T = x.shape
    C = br_wp_f.shape[1]
    nband = N // C
    xt = jnp.transpose(x.reshape(B, nband, C, T), (3, 0, 1, 2)).reshape(T, B * nband, C)
    y1 = _res_rnn(xt, br_gn_g, br_gn_b, br_wih, br_bih, br_whh,
                  br_wp_f, br_wp_b, br_bp)
    xb = jnp.transpose(y1.reshape(T, B, nband, C), (2, 1, 0, 3)).reshape(nband, B * T, C)
    y2 = _res_rnn(xb, bc_gn_g, bc_gn_b, bc_wih, bc_bih, bc_whh,
                  bc_wp_f, bc_wp_b, bc_bp)
    return jnp.transpose(y2.reshape(nband, B, T, C), (1, 0, 3, 2)).reshape(B, N, T)


# speedup vs baseline: 5.3825x; 1.0069x over previous
import functools

import jax
import jax.numpy as jnp
from jax import lax
from jax.experimental import pallas as pl
from jax.experimental.pallas import tpu as pltpu

EPS = float(jnp.finfo(jnp.float32).eps)

MAX_LTB = 16384


def _row_block(bp, l):
    cap = max(8, MAX_LTB // max(l, 1))
    tb = min(bp // 2, cap)
    while tb > 8:
        if bp % tb == 0 and tb % 8 == 0:
            return tb
        tb -= 8
    return min(bp, 8)


def _resrnn_kernel(x_ref, g_ref, b_ref, wf_ref, wb_ref, bihf_ref, bihb_ref,
                   wpf_ref, wpb_ref, bp_ref, o_ref, xn_sc, ob_sc):
    L, TB, C = x_ref.shape
    H = wpf_ref.shape[0]

    x = x_ref[...]
    inv_n = 1.0 / (L * C)
    s1 = jnp.sum(x, axis=0, keepdims=True)
    s2 = jnp.sum(x * x, axis=0, keepdims=True)
    mean = jnp.sum(s1, axis=2, keepdims=True) * inv_n
    ex2 = jnp.sum(s2, axis=2, keepdims=True) * inv_n
    var = ex2 - mean * mean
    a = lax.rsqrt(var + EPS) * g_ref[...]
    bb = b_ref[...] - mean * a
    xn_sc[...] = (x * a + bb).astype(jnp.bfloat16)
    o_ref[...] = x + bp_ref[...]

    w_f = wf_ref[...]
    w_b = wb_ref[...]
    bih_f = bihf_ref[...]
    bih_b = bihb_ref[...]
    wp_f = wpf_ref[...]
    wp_b = wpb_ref[...]

    def sig(v):
        return 0.5 * jnp.tanh(0.5 * v) + 0.5

    def gate(g, c_prev):
        i = sig(g[:, 0:H])
        f = sig(g[:, H:2 * H])
        z = jnp.tanh(g[:, 2 * H:3 * H])
        o = sig(g[:, 3 * H:4 * H])
        c = f * c_prev + i * z
        return o * jnp.tanh(c), c

    def step(t, carry):
        h_f, c_f, h_b, c_b = carry
        zf = jnp.concatenate([xn_sc[t], h_f.astype(jnp.bfloat16)], axis=1)
        zb = jnp.concatenate([xn_sc[L - 1 - t], h_b.astype(jnp.bfloat16)], axis=1)
        pre_f = jnp.dot(zf, w_f, preferred_element_type=jnp.float32) + bih_f
        pre_b = jnp.dot(zb, w_b, preferred_element_type=jnp.float32) + bih_b
        h_f, c_f = gate(pre_f, c_f)
        h_b, c_b = gate(pre_b, c_b)
        o_ref[t] += jnp.dot(h_f.astype(jnp.bfloat16), wp_f,
                            preferred_element_type=jnp.float32)
        ob_sc[L - 1 - t] = jnp.dot(h_b.astype(jnp.bfloat16), wp_b,
                                   preferred_element_type=jnp.float32)
        return h_f, c_f, h_b, c_b

    zeros = jnp.zeros((TB, H), jnp.float32)
    lax.fori_loop(0, L, step, (zeros, zeros, zeros, zeros), unroll=4)
    o_ref[...] += ob_sc[...]


def _res_rnn(xt, gn_g, gn_b, wih, bih, whh, wp_f, wp_b, bp):
    L, Bp, C = xt.shape
    H = wp_f.shape[0]
    G4 = 4 * H
    TB = _row_block(Bp, L)
    grid = (Bp // TB,)

    w_f = jnp.concatenate([wih[:, :G4], whh[:H, :G4]], axis=0)
    w_b = jnp.concatenate([wih[:, G4:], whh[H:, G4:]], axis=0)
    bih_f, bih_b = bih[:, :G4], bih[:, G4:]

    call = pl.pallas_call(
        _resrnn_kernel,
        grid=grid,
        in_specs=[
            pl.BlockSpec((L, TB, C), lambda i: (0, i, 0)),
            pl.BlockSpec((1, 1, C), lambda i: (0, 0, 0)),
            pl.BlockSpec((1, 1, C), lambda i: (0, 0, 0)),
            pl.BlockSpec((C + H, G4), lambda i: (0, 0)),
            pl.BlockSpec((C + H, G4), lambda i: (0, 0)),
            pl.BlockSpec((1, G4), lambda i: (0, 0)),
            pl.BlockSpec((1, G4), lambda i: (0, 0)),
            pl.BlockSpec((H, C), lambda i: (0, 0)),
            pl.BlockSpec((H, C), lambda i: (0, 0)),
            pl.BlockSpec((1, C), lambda i: (0, 0)),
        ],
        out_specs=pl.BlockSpec((L, TB, C), lambda i: (0, i, 0)),
        out_shape=jax.ShapeDtypeStruct((L, Bp, C), jnp.float32),
        input_output_aliases={0: 0},

        scratch_shapes=[
            pltpu.VMEM((L, TB, C), jnp.bfloat16),
            pltpu.VMEM((L, TB, C), jnp.float32),
        ],
        compiler_params=pltpu.CompilerParams(
            dimension_semantics=("parallel",)),
    )
    return call(xt, gn_g, gn_b, w_f, w_b, bih_f, bih_b, wp_f, wp_b, bp)


@functools.partial(jax.jit, static_argnames=())
def kernel(x, br_gn_g, br_gn_b, br_wih, br_bih, br_whh, br_wp_f, br_wp_b, br_bp,
           bc_gn_g, bc_gn_b, bc_wih, bc_bih, bc_whh, bc_wp_f, bc_wp_b, bc_bp):
    B, N, T = x.shape
    C = br_wp_f.shape[1]
    nband = N // C
    xt = jnp.transpose(x.reshape(B, nband, C, T), (3, 0, 1, 2)).reshape(T, B * nband, C)
    y1 = _res_rnn(xt, br_gn_g, br_gn_b, br_wih, br_bih, br_whh,
                  br_wp_f, br_wp_b, br_bp)
    xb = jnp.transpose(y1.reshape(T, B, nband, C), (2, 1, 0, 3)).reshape(nband, B * T, C)
    y2 = _res_rnn(xb, bc_gn_g, bc_gn_b, bc_wih, bc_bih, bc_whh,
                  bc_wp_f, bc_wp_b, bc_bp)
    return jnp.transpose(y2.reshape(nband, B, T, C), (1, 0, 3, 2)).reshape(B, N, T)


# speedup vs baseline: 5.6595x; 1.0515x over previous
import functools

import jax
import jax.numpy as jnp
from jax import lax
from jax.experimental import pallas as pl
from jax.experimental.pallas import tpu as pltpu

EPS = float(jnp.finfo(jnp.float32).eps)

MAX_LTB = 16384


def _row_block(bp, l):
    cap = max(8, MAX_LTB // max(l, 1))
    tb = min(bp // 2, cap)
    while tb > 8:
        if bp % tb == 0 and tb % 8 == 0:
            return tb
        tb -= 8
    return min(bp, 8)


def _resrnn_kernel(x_ref, g_ref, b_ref, wf_ref, wb_ref, bihf_ref, bihb_ref,
                   wpf_ref, wpb_ref, bp_ref, o_ref, xn_sc, ob_sc):
    L, TB, C = x_ref.shape
    H = wpf_ref.shape[0]

    x = x_ref[...]
    inv_n = 1.0 / (L * C)
    s1 = jnp.sum(x, axis=0, keepdims=True)
    s2 = jnp.sum(x * x, axis=0, keepdims=True)
    mean = jnp.sum(s1, axis=2, keepdims=True) * inv_n
    ex2 = jnp.sum(s2, axis=2, keepdims=True) * inv_n
    var = ex2 - mean * mean
    a = lax.rsqrt(var + EPS) * g_ref[...]
    bb = b_ref[...] - mean * a
    xn_sc[...] = (x * a + bb).astype(jnp.bfloat16)
    o_ref[...] = x + bp_ref[...]

    w_f = wf_ref[...]
    w_b = wb_ref[...]
    bih_f = bihf_ref[...]
    bih_b = bihb_ref[...]
    wp_f = wpf_ref[...]
    wp_b = wpb_ref[...]

    def sig(v):
        return 0.5 * jnp.tanh(0.5 * v) + 0.5

    def gate(g, c_prev):
        i = sig(g[:, 0:H])
        f = sig(g[:, H:2 * H])
        z = jnp.tanh(g[:, 2 * H:3 * H])
        o = sig(g[:, 3 * H:4 * H])
        c = f * c_prev + i * z
        return o * jnp.tanh(c), c

    def step(t, carry):
        h_f, c_f, h_b, c_b = carry
        zf = jnp.concatenate([xn_sc[t], h_f.astype(jnp.bfloat16)], axis=1)
        zb = jnp.concatenate([xn_sc[L - 1 - t], h_b.astype(jnp.bfloat16)], axis=1)
        pre_f = jnp.dot(zf, w_f, preferred_element_type=jnp.float32) + bih_f
        pre_b = jnp.dot(zb, w_b, preferred_element_type=jnp.float32) + bih_b
        h_f, c_f = gate(pre_f, c_f)
        h_b, c_b = gate(pre_b, c_b)
        o_ref[t] += jnp.dot(h_f.astype(jnp.bfloat16), wp_f,
                            preferred_element_type=jnp.float32)
        ob_sc[L - 1 - t] = jnp.dot(h_b.astype(jnp.bfloat16), wp_b,
                                   preferred_element_type=jnp.float32)
        return h_f, c_f, h_b, c_b

    zeros = jnp.zeros((TB, H), jnp.float32)
    lax.fori_loop(0, L, step, (zeros, zeros, zeros, zeros), unroll=8)
    o_ref[...] += ob_sc[...]


def _res_rnn(xt, gn_g, gn_b, wih, bih, whh, wp_f, wp_b, bp):
    L, Bp, C = xt.shape
    H = wp_f.shape[0]
    G4 = 4 * H
    TB = _row_block(Bp, L)
    grid = (Bp // TB,)

    w_f = jnp.concatenate([wih[:, :G4], whh[:H, :G4]], axis=0)
    w_b = jnp.concatenate([wih[:, G4:], whh[H:, G4:]], axis=0)
    bih_f, bih_b = bih[:, :G4], bih[:, G4:]

    call = pl.pallas_call(
        _resrnn_kernel,
        grid=grid,
        in_specs=[
            pl.BlockSpec((L, TB, C), lambda i: (0, i, 0)),
            pl.BlockSpec((1, 1, C), lambda i: (0, 0, 0)),
            pl.BlockSpec((1, 1, C), lambda i: (0, 0, 0)),
            pl.BlockSpec((C + H, G4), lambda i: (0, 0)),
            pl.BlockSpec((C + H, G4), lambda i: (0, 0)),
            pl.BlockSpec((1, G4), lambda i: (0, 0)),
            pl.BlockSpec((1, G4), lambda i: (0, 0)),
            pl.BlockSpec((H, C), lambda i: (0, 0)),
            pl.BlockSpec((H, C), lambda i: (0, 0)),
            pl.BlockSpec((1, C), lambda i: (0, 0)),
        ],
        out_specs=pl.BlockSpec((L, TB, C), lambda i: (0, i, 0)),
        out_shape=jax.ShapeDtypeStruct((L, Bp, C), jnp.float32),
        input_output_aliases={0: 0},

        scratch_shapes=[
            pltpu.VMEM((L, TB, C), jnp.bfloat16),
            pltpu.VMEM((L, TB, C), jnp.float32),
        ],
        compiler_params=pltpu.CompilerParams(
            dimension_semantics=("parallel",)),
    )
    return call(xt, gn_g, gn_b, w_f, w_b, bih_f, bih_b, wp_f, wp_b, bp)


@functools.partial(jax.jit, static_argnames=())
def kernel(x, br_gn_g, br_gn_b, br_wih, br_bih, br_whh, br_wp_f, br_wp_b, br_bp,
           bc_gn_g, bc_gn_b, bc_wih, bc_bih, bc_whh, bc_wp_f, bc_wp_b, bc_bp):
    B, N, T = x.shape
    C = br_wp_f.shape[1]
    nband = N // C
    xt = jnp.transpose(x.reshape(B, nband, C, T), (3, 0, 1, 2)).reshape(T, B * nband, C)
    y1 = _res_rnn(xt, br_gn_g, br_gn_b, br_wih, br_bih, br_whh,
                  br_wp_f, br_wp_b, br_bp)
    xb = jnp.transpose(y1.reshape(T, B, nband, C), (2, 1, 0, 3)).reshape(nband, B * T, C)
    y2 = _res_rnn(xb, bc_gn_g, bc_gn_b, bc_wih, bc_bih, bc_whh,
                  bc_wp_f, bc_wp_b, bc_bp)
    return jnp.transpose(y2.reshape(nband, B, T, C), (1, 0, 3, 2)).reshape(B, N, T)
